# baseline (speedup 1.0000x reference)
import jax
import jax.numpy as jnp
from jax import lax
from jax.experimental import pallas as pl
from jax.experimental.pallas import tpu as pltpu

N = 10000
E = 320000
D = 128
NB = 16
RPAD = 10240
EB = 512
FMAX = 3.0e38


def _scale_body(d_ref, bi_ref, o_ref):
    d = d_ref[...]
    bi = bi_ref[...]
    hs = 1.0 / jnp.log(d + 1.0)
    inv = jnp.zeros_like(hs)
    for bb in range(NB):
        m = bi == bb
        ssum = jnp.sum(jnp.where(m, hs, 0.0))
        scnt = jnp.sum(jnp.where(m, 1.0, 0.0))
        inv = inv + jnp.where(m, scnt / ssum, 0.0)
    o_ref[...] = hs * inv


def _scale_stage(deg2d, bi2d):
    return pl.pallas_call(
        _scale_body,
        out_shape=jax.ShapeDtypeStruct((RPAD // D, D), jnp.float32),
    )(deg2d, bi2d)


def _agg_body(ev_ref, t2_ref, h2_ref, sc_ref,
              sum_ref, sq_ref, mx_ref, mn_ref, cnt_ref):
    pid = pl.program_id(0)

    @pl.when(pid == 0)
    def _():
        sum_ref[...] = jnp.zeros((RPAD, D), jnp.float32)
        sq_ref[...] = jnp.zeros((RPAD, D), jnp.float32)
        mx_ref[...] = jnp.full((RPAD, D), -FMAX, jnp.float32)
        mn_ref[...] = jnp.full((RPAD, D), FMAX, jnp.float32)
        cnt_ref[...] = jnp.zeros((RPAD, 16), jnp.float32)

    lane = lax.broadcasted_iota(jnp.int32, (1, D), 1)
    one16 = jnp.ones((1, 16), jnp.float32)

    def body(i, c):
        t = t2_ref[0, 0, i]
        h = h2_ref[0, 0, i]
        r = h >> 7
        cc = h - (r << 7)
        srow = sc_ref[pl.ds(r, 1), :]
        scl = jnp.sum(jnp.where(lane == cc, srow, 0.0))
        v = ev_ref[pl.ds(i, 1), :] * scl
        ts = pl.ds(t, 1)
        sum_ref[ts, :] = sum_ref[ts, :] + v
        sq_ref[ts, :] = sq_ref[ts, :] + v * v
        mx_ref[ts, :] = jnp.maximum(mx_ref[ts, :], v)
        mn_ref[ts, :] = jnp.minimum(mn_ref[ts, :], v)
        cnt_ref[ts, :] = cnt_ref[ts, :] + one16
        return c
    lax.fori_loop(0, EB, body, 0)


def _agg_stage(e_v, t2b, h2b, scale2d):
    whole = pl.BlockSpec((RPAD, D), lambda i: (0, 0))
    smem = lambda: pl.BlockSpec((1, 1, EB), lambda i: (i, 0, 0),
                                memory_space=pltpu.SMEM)
    return pl.pallas_call(
        _agg_body,
        grid=(E // EB,),
        in_specs=[
            pl.BlockSpec((EB, D), lambda i: (i, 0)),
            smem(),
            smem(),
            pl.BlockSpec((RPAD // D, D), lambda i: (0, 0)),
        ],
        out_specs=[whole, whole, whole, whole,
                   pl.BlockSpec((RPAD, 16), lambda i: (0, 0))],
        out_shape=[
            jax.ShapeDtypeStruct((RPAD, D), jnp.float32),
            jax.ShapeDtypeStruct((RPAD, D), jnp.float32),
            jax.ShapeDtypeStruct((RPAD, D), jnp.float32),
            jax.ShapeDtypeStruct((RPAD, D), jnp.float32),
            jax.ShapeDtypeStruct((RPAD, 16), jnp.float32),
        ],
    )(e_v, t2b, h2b, scale2d)


def _proj_body(s_ref, sq_ref, mx_ref, mn_ref, cnt_ref, w_ref, b_ref, o_ref):
    s = s_ref[...]
    cnt = cnt_ref[...][:, 0:1]
    have = cnt > 0
    mean = jnp.where(have, s / jnp.maximum(cnt, 1.0), 0.0)
    mx = jnp.where(have, mx_ref[...], 0.0)
    mn = jnp.where(have, mn_ref[...], 0.0)
    std = jnp.sqrt(jnp.clip(sq_ref[...] - s * s, 1e-8, None))
    agg = jnp.concatenate([mean, mx, mn, std], axis=-1)
    o_ref[...] = lax.dot_general(
        agg, w_ref[...], (((1,), (1,)), ((), ())),
        preferred_element_type=jnp.float32) + b_ref[...]


def _proj_stage(sums, sqs, mxs, mns, cnts, W, b2d):
    BR = 256
    fblk = pl.BlockSpec((BR, D), lambda i: (i, 0))
    return pl.pallas_call(
        _proj_body,
        grid=(RPAD // BR,),
        in_specs=[
            fblk, fblk, fblk, fblk,
            pl.BlockSpec((BR, 16), lambda i: (i, 0)),
            pl.BlockSpec((D, 4 * D), lambda i: (0, 0)),
            pl.BlockSpec((1, D), lambda i: (0, 0)),
        ],
        out_specs=fblk,
        out_shape=jax.ShapeDtypeStruct((RPAD, D), jnp.float32),
    )(sums, sqs, mxs, mns, cnts, W, b2d)


def kernel(o_shape, e_v, h_o_degree, h_batch_idx, N_b, h2pair, t2pair, W, b):
    deg2d = jnp.pad(h_o_degree.astype(jnp.float32), (0, RPAD - N),
                    constant_values=1.0).reshape(RPAD // D, D)
    bi2d = jnp.pad(h_batch_idx.astype(jnp.int32), (0, RPAD - N),
                   constant_values=NB).reshape(RPAD // D, D)
    scale2d = _scale_stage(deg2d, bi2d)

    sums, sqs, mxs, mns, cnts = _agg_stage(
        e_v.astype(jnp.float32),
        t2pair.astype(jnp.int32).reshape(E // EB, 1, EB),
        h2pair.astype(jnp.int32).reshape(E // EB, 1, EB),
        scale2d)

    out = _proj_stage(sums, sqs, mxs, mns, cnts, W.astype(jnp.float32),
                      b.astype(jnp.float32).reshape(1, D))
    return out[:N]


# speedup vs baseline: 5.4068x; 5.4068x over previous
import jax
import jax.numpy as jnp
from jax import lax
from jax.experimental import pallas as pl
from jax.experimental.pallas import tpu as pltpu

N = 10000
E = 320000
D = 128
NB = 16
RPAD = 10240
EB = 512
FMAX = 3.0e38


def _scale_body(d_ref, bi_ref, o_ref):
    d = d_ref[...]
    bi = bi_ref[...]
    hs = 1.0 / jnp.log(d + 1.0)
    inv = jnp.zeros_like(hs)
    for bb in range(NB):
        m = bi == bb
        ssum = jnp.sum(jnp.where(m, hs, 0.0))
        scnt = jnp.sum(jnp.where(m, 1.0, 0.0))
        inv = inv + jnp.where(m, scnt / ssum, 0.0)
    o_ref[...] = hs * inv


def _scale_stage(deg2d, bi2d):
    return pl.pallas_call(
        _scale_body,
        out_shape=jax.ShapeDtypeStruct((RPAD // D, D), jnp.float32),
    )(deg2d, bi2d)


AC = 5 * D


def _agg_body(ev_ref, t2_ref, h2_ref, sc_ref, acc_ref):
    pid = pl.program_id(0)
    lane = lax.broadcasted_iota(jnp.int32, (1, AC), 1)
    m_mx = jnp.logical_and(lane >= 2 * D, lane < 3 * D)
    m_mn = jnp.logical_and(lane >= 3 * D, lane < 4 * D)

    @pl.when(pid == 0)
    def _():
        lane2 = lax.broadcasted_iota(jnp.int32, (RPAD, AC), 1)
        init = jnp.where(
            jnp.logical_and(lane2 >= 2 * D, lane2 < 3 * D), -FMAX,
            jnp.where(jnp.logical_and(lane2 >= 3 * D, lane2 < 4 * D),
                      FMAX, 0.0))
        acc_ref[...] = init

    one128 = jnp.ones((1, D), jnp.float32)

    def body(i, c):
        t = t2_ref[0, 0, i]
        h = h2_ref[0, 0, i]
        r = h >> 7
        scl = sc_ref[r, h - (r << 7)]
        v = ev_ref[pl.ds(i, 1), :] * scl
        vv = jnp.concatenate([v, v * v, v, v, one128], axis=1)
        ts = pl.ds(t, 1)
        u = acc_ref[ts, :]
        acc_ref[ts, :] = jnp.where(
            m_mx, jnp.maximum(u, vv),
            jnp.where(m_mn, jnp.minimum(u, vv), u + vv))
        return c
    lax.fori_loop(0, EB, body, 0)


def _agg_stage(e_v, t2b, h2b, scale2d):
    smem = lambda: pl.BlockSpec((1, 1, EB), lambda i: (i, 0, 0),
                                memory_space=pltpu.SMEM)
    return pl.pallas_call(
        _agg_body,
        grid=(E // EB,),
        in_specs=[
            pl.BlockSpec((EB, D), lambda i: (i, 0)),
            smem(),
            smem(),
            pl.BlockSpec((RPAD // D, D), lambda i: (0, 0),
                         memory_space=pltpu.SMEM),
        ],
        out_specs=pl.BlockSpec((RPAD, AC), lambda i: (0, 0)),
        out_shape=jax.ShapeDtypeStruct((RPAD, AC), jnp.float32),
    )(e_v, t2b, h2b, scale2d)


def _proj_body(acc_ref, w_ref, b_ref, o_ref):
    acc = acc_ref[...]
    s = acc[:, 0:D]
    sq = acc[:, D:2 * D]
    cnt = acc[:, 4 * D:4 * D + 1]
    have = cnt > 0
    mean = jnp.where(have, s / jnp.maximum(cnt, 1.0), 0.0)
    mx = jnp.where(have, acc[:, 2 * D:3 * D], 0.0)
    mn = jnp.where(have, acc[:, 3 * D:4 * D], 0.0)
    std = jnp.sqrt(jnp.clip(sq - s * s, 1e-8, None))
    agg = jnp.concatenate([mean, mx, mn, std], axis=-1)
    o_ref[...] = lax.dot_general(
        agg, w_ref[...], (((1,), (1,)), ((), ())),
        preferred_element_type=jnp.float32) + b_ref[...]


def _proj_stage(acc, W, b2d):
    BR = 256
    return pl.pallas_call(
        _proj_body,
        grid=(RPAD // BR,),
        in_specs=[
            pl.BlockSpec((BR, AC), lambda i: (i, 0)),
            pl.BlockSpec((D, 4 * D), lambda i: (0, 0)),
            pl.BlockSpec((1, D), lambda i: (0, 0)),
        ],
        out_specs=pl.BlockSpec((BR, D), lambda i: (i, 0)),
        out_shape=jax.ShapeDtypeStruct((RPAD, D), jnp.float32),
    )(acc, W, b2d)


def kernel(o_shape, e_v, h_o_degree, h_batch_idx, N_b, h2pair, t2pair, W, b):
    deg2d = jnp.pad(h_o_degree.astype(jnp.float32), (0, RPAD - N),
                    constant_values=1.0).reshape(RPAD // D, D)
    bi2d = jnp.pad(h_batch_idx.astype(jnp.int32), (0, RPAD - N),
                   constant_values=NB).reshape(RPAD // D, D)
    scale2d = _scale_stage(deg2d, bi2d)

    acc = _agg_stage(
        e_v.astype(jnp.float32),
        t2pair.astype(jnp.int32).reshape(E // EB, 1, EB),
        h2pair.astype(jnp.int32).reshape(E // EB, 1, EB),
        scale2d)

    out = _proj_stage(acc, W.astype(jnp.float32),
                      b.astype(jnp.float32).reshape(1, D))
    return out[:N]


# speedup vs baseline: 9.7718x; 1.8073x over previous
import jax
import jax.numpy as jnp
from jax import lax
from jax.experimental import pallas as pl
from jax.experimental.pallas import tpu as pltpu

N = 10000
E = 320000
D = 128
NB = 16
RPAD = 10240
EB = 512
FMAX = 3.0e38


def _scale_body(d_ref, bi_ref, o_ref):
    d = d_ref[...]
    bi = bi_ref[...]
    hs = 1.0 / jnp.log(d + 1.0)
    inv = jnp.zeros_like(hs)
    for bb in range(NB):
        m = bi == bb
        ssum = jnp.sum(jnp.where(m, hs, 0.0))
        scnt = jnp.sum(jnp.where(m, 1.0, 0.0))
        inv = inv + jnp.where(m, scnt / ssum, 0.0)
    o_ref[...] = hs * inv


def _scale_stage(deg2d, bi2d):
    return pl.pallas_call(
        _scale_body,
        out_shape=jax.ShapeDtypeStruct((RPAD // D, D), jnp.float32),
    )(deg2d, bi2d)


AC = 5 * D


def _agg_body(ev_ref, t2_ref, h2_ref, sc_ref, acc0_ref, acc1_ref):
    pid = pl.program_id(0)
    lane = lax.broadcasted_iota(jnp.int32, (1, AC), 1)
    m_mx = jnp.logical_and(lane >= 2 * D, lane < 3 * D)
    m_mn = jnp.logical_and(lane >= 3 * D, lane < 4 * D)

    @pl.when(pid == 0)
    def _():
        lane2 = lax.broadcasted_iota(jnp.int32, (RPAD, AC), 1)
        init = jnp.where(
            jnp.logical_and(lane2 >= 2 * D, lane2 < 3 * D), -FMAX,
            jnp.where(jnp.logical_and(lane2 >= 3 * D, lane2 < 4 * D),
                      FMAX, 0.0))
        acc0_ref[...] = init
        acc1_ref[...] = init

    one128 = jnp.ones((1, D), jnp.float32)

    def upd(acc_ref, i):
        t = t2_ref[0, 0, i]
        h = h2_ref[0, 0, i]
        r = h >> 7
        scl = sc_ref[r, h - (r << 7)]
        v = ev_ref[pl.ds(i, 1), :] * scl
        vv = jnp.concatenate([v, v * v, v, v, one128], axis=1)
        ts = pl.ds(t, 1)
        u = acc_ref[ts, :]
        acc_ref[ts, :] = jnp.where(
            m_mx, jnp.maximum(u, vv),
            jnp.where(m_mn, jnp.minimum(u, vv), u + vv))

    def body(j, c):
        upd(acc0_ref, 2 * j)
        upd(acc1_ref, 2 * j + 1)
        return c
    lax.fori_loop(0, EB // 2, body, 0)


def _agg_stage(e_v, t2b, h2b, scale2d):
    smem = lambda: pl.BlockSpec((1, 1, EB), lambda i: (i, 0, 0),
                                memory_space=pltpu.SMEM)
    whole = pl.BlockSpec((RPAD, AC), lambda i: (0, 0))
    return pl.pallas_call(
        _agg_body,
        grid=(E // EB,),
        in_specs=[
            pl.BlockSpec((EB, D), lambda i: (i, 0)),
            smem(),
            smem(),
            pl.BlockSpec((RPAD // D, D), lambda i: (0, 0),
                         memory_space=pltpu.SMEM),
        ],
        out_specs=[whole, whole],
        out_shape=[
            jax.ShapeDtypeStruct((RPAD, AC), jnp.float32),
            jax.ShapeDtypeStruct((RPAD, AC), jnp.float32),
        ],
    )(e_v, t2b, h2b, scale2d)


def _proj_body(acc0_ref, acc1_ref, w_ref, b_ref, o_ref):
    a0 = acc0_ref[...]
    a1 = acc1_ref[...]
    s = a0[:, 0:D] + a1[:, 0:D]
    sq = a0[:, D:2 * D] + a1[:, D:2 * D]
    cnt = a0[:, 4 * D:4 * D + 1] + a1[:, 4 * D:4 * D + 1]
    have = cnt > 0
    mean = jnp.where(have, s / jnp.maximum(cnt, 1.0), 0.0)
    mx = jnp.where(have,
                   jnp.maximum(a0[:, 2 * D:3 * D], a1[:, 2 * D:3 * D]), 0.0)
    mn = jnp.where(have,
                   jnp.minimum(a0[:, 3 * D:4 * D], a1[:, 3 * D:4 * D]), 0.0)
    std = jnp.sqrt(jnp.clip(sq - s * s, 1e-8, None))
    agg = jnp.concatenate([mean, mx, mn, std], axis=-1)
    o_ref[...] = lax.dot_general(
        agg, w_ref[...], (((1,), (1,)), ((), ())),
        preferred_element_type=jnp.float32) + b_ref[...]


def _proj_stage(acc0, acc1, W, b2d):
    BR = 256
    ablk = pl.BlockSpec((BR, AC), lambda i: (i, 0))
    return pl.pallas_call(
        _proj_body,
        grid=(RPAD // BR,),
        in_specs=[
            ablk, ablk,
            pl.BlockSpec((D, 4 * D), lambda i: (0, 0)),
            pl.BlockSpec((1, D), lambda i: (0, 0)),
        ],
        out_specs=pl.BlockSpec((BR, D), lambda i: (i, 0)),
        out_shape=jax.ShapeDtypeStruct((RPAD, D), jnp.float32),
    )(acc0, acc1, W, b2d)


def kernel(o_shape, e_v, h_o_degree, h_batch_idx, N_b, h2pair, t2pair, W, b):
    deg2d = jnp.pad(h_o_degree.astype(jnp.float32), (0, RPAD - N),
                    constant_values=1.0).reshape(RPAD // D, D)
    bi2d = jnp.pad(h_batch_idx.astype(jnp.int32), (0, RPAD - N),
                   constant_values=NB).reshape(RPAD // D, D)
    scale2d = _scale_stage(deg2d, bi2d)

    acc0, acc1 = _agg_stage(
        e_v.astype(jnp.float32),
        t2pair.astype(jnp.int32).reshape(E // EB, 1, EB),
        h2pair.astype(jnp.int32).reshape(E // EB, 1, EB),
        scale2d)

    out = _proj_stage(acc0, acc1, W.astype(jnp.float32),
                      b.astype(jnp.float32).reshape(1, D))
    return out[:N]
